# baseline (speedup 1.0000x reference)
import functools

import jax
import jax.numpy as jnp
from jax.experimental import pallas as pl
from jax.experimental.pallas import tpu as pltpu

DO = 128
G = 8
_GRID = [-2.0 + i * (4.0 / 7.0) for i in range(G)]
_INV_DENOM = 7.0 / 4.0
_PREC = jax.lax.Precision.DEFAULT


def _dot(a, b):
    return jnp.dot(a, b, preferred_element_type=jnp.float32, precision=_PREC)


def _kan_block(x, g_row, b_row, wg_ref, cast_bf16=False):
    mu = jnp.mean(x, axis=-1, keepdims=True)
    var = jnp.mean((x - mu) ** 2, axis=-1, keepdims=True)
    xn = (x - mu) / jnp.sqrt(var + 1e-5) * g_row + b_row
    if cast_bf16:
        xn = xn.astype(jnp.bfloat16)
    d = x.shape[-1]
    acc = jnp.zeros((x.shape[0], d), jnp.float32)
    for gi in range(G):
        basis = 1.0 - jnp.tanh((xn - _GRID[gi]) * _INV_DENOM) ** 2
        acc = acc + _dot(basis, wg_ref[gi])
    return acc


def _att_block(e1, e2, w_ref, uT_ref):
    v1 = jnp.tanh(_dot(e1, w_ref[...]))
    v2 = jnp.tanh(_dot(e2, w_ref[...]))
    u = uT_ref[0:1, :]
    s1 = jnp.sum(v1 * u, axis=-1, keepdims=True)
    s2 = jnp.sum(v2 * u, axis=-1, keepdims=True)
    m = jnp.maximum(s1, s2)
    x1 = jnp.exp(s1 - m)
    x2 = jnp.exp(s2 - m)
    z = x1 + x2
    return (x1 / z) * e1 + (x2 / z) * e2


def _ff_block(x, w1_ref, b1_ref, w2_ref, b2_ref):
    h = jax.nn.gelu(_dot(x, w1_ref[...]) + b1_ref[0:1, :])
    return _dot(h, w2_ref[...]) + b2_ref[0:1, :]


def _encoder_body(asp1, aft1, asp2, aft2, asp3, aft3,
                  f1, f2, f3, we1, we2, we3,
                  wg1, wg2, wg3, g1, b1, g2, b2, g3, b3,
                  aw1, au1, aw2, au2, aw3, au3,
                  gwT, gb,
                  e1w1, e1b1, e1w2, e1b2,
                  e2w1, e2b1, e2w2, e2b2,
                  e3w1, e3b1, e3w2, e3b2,
                  out_ref,
                  p1, p2, p3,
                  s_sp1, s_ft1, s_sp2, s_ft2, s_sp3, s_ft3):
    i = pl.program_id(0)
    k = pl.program_id(1)
    nk = pl.num_programs(1)
    kb = asp1.shape[1]
    accs = (s_sp1, s_ft1, s_sp2, s_ft2, s_sp3, s_ft3)
    adjs = (asp1, aft1, asp2, aft2, asp3, aft3)
    ps = (p1, p1, p2, p2, p3, p3)

    @pl.when(jnp.logical_and(i == 0, k == 0))
    def _project():
        p1[...] = _dot(f1[...], we1[...])
        p2[...] = _dot(f2[...], we2[...])
        p3[...] = _dot(f3[...], we3[...])

    @pl.when(k == 0)
    def _init():
        for acc, adj, p in zip(accs, adjs, ps):
            acc[...] = _dot(adj[...], p[pl.ds(k * kb, kb), :])

    @pl.when(k > 0)
    def _accum():
        for acc, adj, p in zip(accs, adjs, ps):
            acc[...] = acc[...] + _dot(adj[...], p[pl.ds(k * kb, kb), :])

    @pl.when(k == nk - 1)
    def _epilogue():
        l1 = _att_block(
            _kan_block(s_sp1[...], g1[0:1, :], b1[0:1, :], wg1),
            _kan_block(s_ft1[...], g1[0:1, :], b1[0:1, :], wg1),
            aw1, au1)
        l2 = _att_block(
            _kan_block(s_sp2[...], g2[0:1, :], b2[0:1, :], wg2),
            _kan_block(s_ft2[...], g2[0:1, :], b2[0:1, :], wg2),
            aw2, au2)
        l3 = _att_block(
            _kan_block(s_sp3[...], g3[0:1, :], b3[0:1, :], wg3),
            _kan_block(s_ft3[...], g3[0:1, :], b3[0:1, :], wg3),
            aw3, au3)

        gate_in = (l1 + l2 + l3) * (1.0 / 3.0)
        s0 = jnp.sum(gate_in * gwT[0:1, :], axis=-1, keepdims=True) + gb[0, 0]
        s1 = jnp.sum(gate_in * gwT[1:2, :], axis=-1, keepdims=True) + gb[0, 1]
        s2 = jnp.sum(gate_in * gwT[2:3, :], axis=-1, keepdims=True) + gb[0, 2]
        m = jnp.maximum(jnp.maximum(s0, s1), s2)
        x0, x1, x2 = jnp.exp(s0 - m), jnp.exp(s1 - m), jnp.exp(s2 - m)
        z = x0 + x1 + x2
        gs0, gs1, gs2 = x0 / z, x1 / z, x2 / z

        o0 = _ff_block(l1, e1w1, e1b1, e1w2, e1b2)
        o1 = _ff_block(l2, e2w1, e2b1, e2w2, e2b2)
        o2 = _ff_block(l3, e3w1, e3b1, e3w2, e3b2)

        m0 = (gs0 >= 0.3).astype(jnp.float32)
        m1 = (gs1 >= 0.3).astype(jnp.float32)
        m2 = (gs2 >= 0.3).astype(jnp.float32)
        ms0, ms1, ms2 = gs0 * m0, gs1 * m1, gs2 * m2
        denom = ms0 + ms1 + ms2 + 1e-6
        fused = (ms0 * o0 + ms1 * o1 + ms2 * o2) / denom

        is0 = jnp.logical_and(gs0 >= gs1, gs0 >= gs2).astype(jnp.float32)
        is1 = (1.0 - is0) * (gs1 >= gs2).astype(jnp.float32)
        is2 = (1.0 - is0) * (1.0 - is1)
        fb = is0 * o0 + is1 * o1 + is2 * o2
        fbm = (m0 + m1 + m2 == 0.0).astype(jnp.float32)
        out_ref[...] = fbm * fb + (1.0 - fbm) * fused


def _decoder_body(asp1, asp2, asp3, fused_ref,
                  wd1, wd2, wd3,
                  gd1, bd1, gd2, bd2, gd3, bd3,
                  wgd1, wgd2, wgd3,
                  out_ref,
                  t1s, t2s, t3s, *, dims):
    d1, d2, d3 = dims
    i = pl.program_id(0)
    k = pl.program_id(1)
    nk = pl.num_programs(1)
    kb = asp1.shape[1]
    fb = fused_ref[pl.ds(k * kb, kb), :]

    @pl.when(k == 0)
    def _init():
        t1s[...] = _dot(asp1[...], fb)
        t2s[...] = _dot(asp2[...], fb)
        t3s[...] = _dot(asp3[...], fb)

    @pl.when(k > 0)
    def _accum():
        t1s[...] = t1s[...] + _dot(asp1[...], fb)
        t2s[...] = t2s[...] + _dot(asp2[...], fb)
        t3s[...] = t3s[...] + _dot(asp3[...], fb)

    @pl.when(k == nk - 1)
    def _epilogue():
        rb = out_ref.shape[0]
        r1 = _kan_block(_dot(t1s[...], wd1[...]), gd1[0:1, :], bd1[0:1, :], wgd1, True)
        r2 = _kan_block(_dot(t2s[...], wd2[...]), gd2[0:1, :], bd2[0:1, :], wgd2, True)
        r3 = _kan_block(_dot(t3s[...], wd3[...]), gd3[0:1, :], bd3[0:1, :], wgd3, True)
        out_ref[:, 0:DO] = fused_ref[pl.ds(i * rb, rb), :]
        out_ref[:, DO:DO + d1] = r1
        out_ref[:, DO + d1:DO + d1 + d2] = r2
        out_ref[:, DO + d1 + d2:DO + d1 + d2 + d3] = r3


def _full_spec(shape):
    nd = len(shape)
    return pl.BlockSpec(shape, lambda i, k, _nd=nd: (0,) * _nd)


def _wg_stack(kan_w, d):
    return kan_w.T.reshape(d, G, d).transpose(1, 0, 2)


def kernel(features_omics1, features_omics2, features_omics3, adj_spatial_omics1, adj_feature_omics1, adj_spatial_omics2, adj_feature_omics2, adj_spatial_omics3, adj_feature_omics3, W_e1, W_e2, W_e3, W_d1, W_d2, W_d3, kan_e1_g, kan_e1_b, kan_e1_w, kan_e2_g, kan_e2_b, kan_e2_w, kan_e3_g, kan_e3_b, kan_e3_w, kan_d1_g, kan_d1_b, kan_d1_w, kan_d2_g, kan_d2_b, kan_d2_w, kan_d3_g, kan_d3_b, kan_d3_w, att1_w, att1_u, att2_w, att2_u, att3_w, att3_u, gate_w, gate_b, exp1_w1, exp1_b1, exp1_w2, exp1_b2, exp2_w1, exp2_b1, exp2_w2, exp2_b2, exp3_w1, exp3_b1, exp3_w2, exp3_b2):
    n = adj_spatial_omics1.shape[0]
    d1 = features_omics1.shape[1]
    d2 = features_omics2.shape[1]
    d3 = features_omics3.shape[1]

    rb = 256 if n % 256 == 0 else n
    kb = 1024 if n % 1024 == 0 else n
    f32 = jnp.float32

    bf16 = jnp.bfloat16
    wg_e1 = _wg_stack(kan_e1_w, DO)
    wg_e2 = _wg_stack(kan_e2_w, DO)
    wg_e3 = _wg_stack(kan_e3_w, DO)
    wg_d1 = _wg_stack(kan_d1_w, d1).astype(bf16)
    wg_d2 = _wg_stack(kan_d2_w, d2).astype(bf16)
    wg_d3 = _wg_stack(kan_d3_w, d3).astype(bf16)

    row = lambda v: v.reshape(1, -1)
    uT1, uT2, uT3 = att1_u.reshape(1, DO), att2_u.reshape(1, DO), att3_u.reshape(1, DO)
    gwT = jnp.zeros((8, DO), f32).at[0:3, :].set(gate_w.T)
    gb = jnp.zeros((1, DO), f32).at[0, 0:3].set(gate_b)

    adj_spec = pl.BlockSpec((rb, kb), lambda i, k: (i, k))
    fused = pl.pallas_call(
        _encoder_body,
        grid=(n // rb, n // kb),
        in_specs=[
            adj_spec, adj_spec, adj_spec, adj_spec, adj_spec, adj_spec,
            _full_spec((n, d1)), _full_spec((n, d2)), _full_spec((n, d3)),
            _full_spec((d1, DO)), _full_spec((d2, DO)), _full_spec((d3, DO)),
            _full_spec((G, DO, DO)), _full_spec((G, DO, DO)), _full_spec((G, DO, DO)),
            _full_spec((1, DO)), _full_spec((1, DO)),
            _full_spec((1, DO)), _full_spec((1, DO)),
            _full_spec((1, DO)), _full_spec((1, DO)),
            _full_spec((DO, DO)), _full_spec((1, DO)),
            _full_spec((DO, DO)), _full_spec((1, DO)),
            _full_spec((DO, DO)), _full_spec((1, DO)),
            _full_spec((8, DO)), _full_spec((1, DO)),
            _full_spec((DO, 4 * DO)), _full_spec((1, 4 * DO)),
            _full_spec((4 * DO, DO)), _full_spec((1, DO)),
            _full_spec((DO, 4 * DO)), _full_spec((1, 4 * DO)),
            _full_spec((4 * DO, DO)), _full_spec((1, DO)),
            _full_spec((DO, 4 * DO)), _full_spec((1, 4 * DO)),
            _full_spec((4 * DO, DO)), _full_spec((1, DO)),
        ],
        out_specs=pl.BlockSpec((rb, DO), lambda i, k: (i, 0)),
        out_shape=jax.ShapeDtypeStruct((n, DO), f32),
        scratch_shapes=([pltpu.VMEM((n, DO), f32)] * 3
                        + [pltpu.VMEM((rb, DO), f32)] * 6),
    )(adj_spatial_omics1, adj_feature_omics1, adj_spatial_omics2,
      adj_feature_omics2, adj_spatial_omics3, adj_feature_omics3,
      features_omics1, features_omics2, features_omics3,
      W_e1, W_e2, W_e3,
      wg_e1, wg_e2, wg_e3,
      row(kan_e1_g), row(kan_e1_b), row(kan_e2_g), row(kan_e2_b),
      row(kan_e3_g), row(kan_e3_b),
      att1_w, uT1, att2_w, uT2, att3_w, uT3,
      gwT, gb,
      exp1_w1, row(exp1_b1), exp1_w2, row(exp1_b2),
      exp2_w1, row(exp2_b1), exp2_w2, row(exp2_b2),
      exp3_w1, row(exp3_b1), exp3_w2, row(exp3_b2))

    dtot = DO + d1 + d2 + d3
    out = pl.pallas_call(
        functools.partial(_decoder_body, dims=(d1, d2, d3)),
        grid=(n // rb, n // kb),
        in_specs=[
            adj_spec, adj_spec, adj_spec,
            _full_spec((n, DO)),
            _full_spec((DO, d1)), _full_spec((DO, d2)), _full_spec((DO, d3)),
            _full_spec((1, d1)), _full_spec((1, d1)),
            _full_spec((1, d2)), _full_spec((1, d2)),
            _full_spec((1, d3)), _full_spec((1, d3)),
            _full_spec((G, d1, d1)), _full_spec((G, d2, d2)), _full_spec((G, d3, d3)),
        ],
        out_specs=pl.BlockSpec((rb, dtot), lambda i, k: (i, 0)),
        out_shape=jax.ShapeDtypeStruct((n, dtot), f32),
        scratch_shapes=[pltpu.VMEM((rb, DO), f32)] * 3,
    )(adj_spatial_omics1, adj_spatial_omics2, adj_spatial_omics3,
      fused,
      W_d1, W_d2, W_d3,
      row(kan_d1_g), row(kan_d1_b), row(kan_d2_g), row(kan_d2_b),
      row(kan_d3_g), row(kan_d3_b),
      wg_d1, wg_d2, wg_d3)

    return out


# speedup vs baseline: 1.1566x; 1.1566x over previous
import functools

import jax
import jax.numpy as jnp
from jax.experimental import pallas as pl
from jax.experimental.pallas import tpu as pltpu

DO = 128
G = 8
_GRID = [-2.0 + i * (4.0 / 7.0) for i in range(G)]
_INV_DENOM = 7.0 / 4.0
_PREC = jax.lax.Precision.DEFAULT


def _dot(a, b):
    return jnp.dot(a, b, preferred_element_type=jnp.float32, precision=_PREC)


def _kan_block(x, g_row, b_row, wg_ref, cast_bf16=False):
    mu = jnp.mean(x, axis=-1, keepdims=True)
    var = jnp.mean((x - mu) ** 2, axis=-1, keepdims=True)
    xn = (x - mu) / jnp.sqrt(var + 1e-5) * g_row + b_row
    if cast_bf16:
        xn = xn.astype(jnp.bfloat16)
    d = x.shape[-1]
    acc = jnp.zeros((x.shape[0], d), jnp.float32)
    for gi in range(G):
        basis = 1.0 - jnp.tanh((xn - _GRID[gi]) * _INV_DENOM) ** 2
        acc = acc + _dot(basis, wg_ref[gi])
    return acc


def _att_block(e1, e2, w_ref, uT_ref):
    v1 = jnp.tanh(_dot(e1, w_ref[...]))
    v2 = jnp.tanh(_dot(e2, w_ref[...]))
    u = uT_ref[0:1, :]
    s1 = jnp.sum(v1 * u, axis=-1, keepdims=True)
    s2 = jnp.sum(v2 * u, axis=-1, keepdims=True)
    m = jnp.maximum(s1, s2)
    x1 = jnp.exp(s1 - m)
    x2 = jnp.exp(s2 - m)
    z = x1 + x2
    return (x1 / z) * e1 + (x2 / z) * e2


def _ff_block(x, w1_ref, b1_ref, w2_ref, b2_ref):
    h = jax.nn.gelu(_dot(x, w1_ref[...]) + b1_ref[0:1, :])
    return _dot(h, w2_ref[...]) + b2_ref[0:1, :]


def _encoder_body(asp1, aft1, asp2, aft2, asp3, aft3,
                  f1, f2, f3, we1, we2, we3,
                  wg1, wg2, wg3, g1, b1, g2, b2, g3, b3,
                  aw1, au1, aw2, au2, aw3, au3,
                  gwT, gb,
                  e1w1, e1b1, e1w2, e1b2,
                  e2w1, e2b1, e2w2, e2b2,
                  e3w1, e3b1, e3w2, e3b2,
                  out_ref,
                  p1, p2, p3,
                  s_sp1, s_ft1, s_sp2, s_ft2, s_sp3, s_ft3):
    i = pl.program_id(0)
    k = pl.program_id(1)
    nk = pl.num_programs(1)
    kb = asp1.shape[1]
    accs = (s_sp1, s_ft1, s_sp2, s_ft2, s_sp3, s_ft3)
    adjs = (asp1, aft1, asp2, aft2, asp3, aft3)
    ps = (p1, p1, p2, p2, p3, p3)

    @pl.when(jnp.logical_and(i == 0, k == 0))
    def _project():
        p1[...] = _dot(f1[...], we1[...])
        p2[...] = _dot(f2[...], we2[...])
        p3[...] = _dot(f3[...], we3[...])

    @pl.when(k == 0)
    def _init():
        for acc, adj, p in zip(accs, adjs, ps):
            acc[...] = _dot(adj[...], p[pl.ds(k * kb, kb), :])

    @pl.when(k > 0)
    def _accum():
        for acc, adj, p in zip(accs, adjs, ps):
            acc[...] = acc[...] + _dot(adj[...], p[pl.ds(k * kb, kb), :])

    @pl.when(k == nk - 1)
    def _epilogue():
        l1 = _att_block(
            _kan_block(s_sp1[...], g1[0:1, :], b1[0:1, :], wg1),
            _kan_block(s_ft1[...], g1[0:1, :], b1[0:1, :], wg1),
            aw1, au1)
        l2 = _att_block(
            _kan_block(s_sp2[...], g2[0:1, :], b2[0:1, :], wg2),
            _kan_block(s_ft2[...], g2[0:1, :], b2[0:1, :], wg2),
            aw2, au2)
        l3 = _att_block(
            _kan_block(s_sp3[...], g3[0:1, :], b3[0:1, :], wg3),
            _kan_block(s_ft3[...], g3[0:1, :], b3[0:1, :], wg3),
            aw3, au3)

        gate_in = (l1 + l2 + l3) * (1.0 / 3.0)
        s0 = jnp.sum(gate_in * gwT[0:1, :], axis=-1, keepdims=True) + gb[0, 0]
        s1 = jnp.sum(gate_in * gwT[1:2, :], axis=-1, keepdims=True) + gb[0, 1]
        s2 = jnp.sum(gate_in * gwT[2:3, :], axis=-1, keepdims=True) + gb[0, 2]
        m = jnp.maximum(jnp.maximum(s0, s1), s2)
        x0, x1, x2 = jnp.exp(s0 - m), jnp.exp(s1 - m), jnp.exp(s2 - m)
        z = x0 + x1 + x2
        gs0, gs1, gs2 = x0 / z, x1 / z, x2 / z

        o0 = _ff_block(l1, e1w1, e1b1, e1w2, e1b2)
        o1 = _ff_block(l2, e2w1, e2b1, e2w2, e2b2)
        o2 = _ff_block(l3, e3w1, e3b1, e3w2, e3b2)

        m0 = (gs0 >= 0.3).astype(jnp.float32)
        m1 = (gs1 >= 0.3).astype(jnp.float32)
        m2 = (gs2 >= 0.3).astype(jnp.float32)
        ms0, ms1, ms2 = gs0 * m0, gs1 * m1, gs2 * m2
        denom = ms0 + ms1 + ms2 + 1e-6
        fused = (ms0 * o0 + ms1 * o1 + ms2 * o2) / denom

        is0 = jnp.logical_and(gs0 >= gs1, gs0 >= gs2).astype(jnp.float32)
        is1 = (1.0 - is0) * (gs1 >= gs2).astype(jnp.float32)
        is2 = (1.0 - is0) * (1.0 - is1)
        fb = is0 * o0 + is1 * o1 + is2 * o2
        fbm = (m0 + m1 + m2 == 0.0).astype(jnp.float32)
        out_ref[...] = fbm * fb + (1.0 - fbm) * fused


def _decoder_body(asp1, asp2, asp3, fused_ref,
                  wd1, wd2, wd3,
                  gd1, bd1, gd2, bd2, gd3, bd3,
                  wgd1, wgd2, wgd3,
                  out_ref,
                  t1s, t2s, t3s, *, dims):
    d1, d2, d3 = dims
    i = pl.program_id(0)
    k = pl.program_id(1)
    nk = pl.num_programs(1)
    kb = asp1.shape[1]
    fb = fused_ref[pl.ds(k * kb, kb), :]

    @pl.when(k == 0)
    def _init():
        t1s[...] = _dot(asp1[...], fb)
        t2s[...] = _dot(asp2[...], fb)
        t3s[...] = _dot(asp3[...], fb)

    @pl.when(k > 0)
    def _accum():
        t1s[...] = t1s[...] + _dot(asp1[...], fb)
        t2s[...] = t2s[...] + _dot(asp2[...], fb)
        t3s[...] = t3s[...] + _dot(asp3[...], fb)

    @pl.when(k == nk - 1)
    def _epilogue():
        rb = out_ref.shape[0]
        r1 = _kan_block(_dot(t1s[...], wd1[...]), gd1[0:1, :], bd1[0:1, :], wgd1, True)
        r2 = _kan_block(_dot(t2s[...], wd2[...]), gd2[0:1, :], bd2[0:1, :], wgd2, True)
        r3 = _kan_block(_dot(t3s[...], wd3[...]), gd3[0:1, :], bd3[0:1, :], wgd3, True)
        out_ref[:, 0:DO] = fused_ref[pl.ds(i * rb, rb), :]
        out_ref[:, DO:DO + d1] = r1
        out_ref[:, DO + d1:DO + d1 + d2] = r2
        out_ref[:, DO + d1 + d2:DO + d1 + d2 + d3] = r3


def _full_spec(shape):
    nd = len(shape)
    return pl.BlockSpec(shape, lambda i, k, _nd=nd: (0,) * _nd)


def _wg_stack(kan_w, d):
    return kan_w.T.reshape(d, G, d).transpose(1, 0, 2)


def kernel(features_omics1, features_omics2, features_omics3, adj_spatial_omics1, adj_feature_omics1, adj_spatial_omics2, adj_feature_omics2, adj_spatial_omics3, adj_feature_omics3, W_e1, W_e2, W_e3, W_d1, W_d2, W_d3, kan_e1_g, kan_e1_b, kan_e1_w, kan_e2_g, kan_e2_b, kan_e2_w, kan_e3_g, kan_e3_b, kan_e3_w, kan_d1_g, kan_d1_b, kan_d1_w, kan_d2_g, kan_d2_b, kan_d2_w, kan_d3_g, kan_d3_b, kan_d3_w, att1_w, att1_u, att2_w, att2_u, att3_w, att3_u, gate_w, gate_b, exp1_w1, exp1_b1, exp1_w2, exp1_b2, exp2_w1, exp2_b1, exp2_w2, exp2_b2, exp3_w1, exp3_b1, exp3_w2, exp3_b2):
    n = adj_spatial_omics1.shape[0]
    d1 = features_omics1.shape[1]
    d2 = features_omics2.shape[1]
    d3 = features_omics3.shape[1]

    rb = 1024 if n % 1024 == 0 else n
    kb = 512 if n % 512 == 0 else n
    f32 = jnp.float32

    bf16 = jnp.bfloat16
    wg_e1 = _wg_stack(kan_e1_w, DO)
    wg_e2 = _wg_stack(kan_e2_w, DO)
    wg_e3 = _wg_stack(kan_e3_w, DO)
    wg_d1 = _wg_stack(kan_d1_w, d1).astype(bf16)
    wg_d2 = _wg_stack(kan_d2_w, d2).astype(bf16)
    wg_d3 = _wg_stack(kan_d3_w, d3).astype(bf16)

    row = lambda v: v.reshape(1, -1)
    uT1, uT2, uT3 = att1_u.reshape(1, DO), att2_u.reshape(1, DO), att3_u.reshape(1, DO)
    gwT = jnp.zeros((8, DO), f32).at[0:3, :].set(gate_w.T)
    gb = jnp.zeros((1, DO), f32).at[0, 0:3].set(gate_b)

    adj_spec = pl.BlockSpec((rb, kb), lambda i, k: (i, k))
    fused = pl.pallas_call(
        _encoder_body,
        grid=(n // rb, n // kb),
        in_specs=[
            adj_spec, adj_spec, adj_spec, adj_spec, adj_spec, adj_spec,
            _full_spec((n, d1)), _full_spec((n, d2)), _full_spec((n, d3)),
            _full_spec((d1, DO)), _full_spec((d2, DO)), _full_spec((d3, DO)),
            _full_spec((G, DO, DO)), _full_spec((G, DO, DO)), _full_spec((G, DO, DO)),
            _full_spec((1, DO)), _full_spec((1, DO)),
            _full_spec((1, DO)), _full_spec((1, DO)),
            _full_spec((1, DO)), _full_spec((1, DO)),
            _full_spec((DO, DO)), _full_spec((1, DO)),
            _full_spec((DO, DO)), _full_spec((1, DO)),
            _full_spec((DO, DO)), _full_spec((1, DO)),
            _full_spec((8, DO)), _full_spec((1, DO)),
            _full_spec((DO, 4 * DO)), _full_spec((1, 4 * DO)),
            _full_spec((4 * DO, DO)), _full_spec((1, DO)),
            _full_spec((DO, 4 * DO)), _full_spec((1, 4 * DO)),
            _full_spec((4 * DO, DO)), _full_spec((1, DO)),
            _full_spec((DO, 4 * DO)), _full_spec((1, 4 * DO)),
            _full_spec((4 * DO, DO)), _full_spec((1, DO)),
        ],
        out_specs=pl.BlockSpec((rb, DO), lambda i, k: (i, 0)),
        out_shape=jax.ShapeDtypeStruct((n, DO), f32),
        scratch_shapes=([pltpu.VMEM((n, DO), f32)] * 3
                        + [pltpu.VMEM((rb, DO), f32)] * 6),
    )(adj_spatial_omics1, adj_feature_omics1, adj_spatial_omics2,
      adj_feature_omics2, adj_spatial_omics3, adj_feature_omics3,
      features_omics1, features_omics2, features_omics3,
      W_e1, W_e2, W_e3,
      wg_e1, wg_e2, wg_e3,
      row(kan_e1_g), row(kan_e1_b), row(kan_e2_g), row(kan_e2_b),
      row(kan_e3_g), row(kan_e3_b),
      att1_w, uT1, att2_w, uT2, att3_w, uT3,
      gwT, gb,
      exp1_w1, row(exp1_b1), exp1_w2, row(exp1_b2),
      exp2_w1, row(exp2_b1), exp2_w2, row(exp2_b2),
      exp3_w1, row(exp3_b1), exp3_w2, row(exp3_b2))

    dtot = DO + d1 + d2 + d3
    out = pl.pallas_call(
        functools.partial(_decoder_body, dims=(d1, d2, d3)),
        grid=(n // rb, n // kb),
        in_specs=[
            adj_spec, adj_spec, adj_spec,
            _full_spec((n, DO)),
            _full_spec((DO, d1)), _full_spec((DO, d2)), _full_spec((DO, d3)),
            _full_spec((1, d1)), _full_spec((1, d1)),
            _full_spec((1, d2)), _full_spec((1, d2)),
            _full_spec((1, d3)), _full_spec((1, d3)),
            _full_spec((G, d1, d1)), _full_spec((G, d2, d2)), _full_spec((G, d3, d3)),
        ],
        out_specs=pl.BlockSpec((rb, dtot), lambda i, k: (i, 0)),
        out_shape=jax.ShapeDtypeStruct((n, dtot), f32),
        scratch_shapes=[pltpu.VMEM((rb, DO), f32)] * 3,
    )(adj_spatial_omics1, adj_spatial_omics2, adj_spatial_omics3,
      fused,
      W_d1, W_d2, W_d3,
      row(kan_d1_g), row(kan_d1_b), row(kan_d2_g), row(kan_d2_b),
      row(kan_d3_g), row(kan_d3_b),
      wg_d1, wg_d2, wg_d3)

    return out
